# baseline (speedup 1.0000x reference)
import functools
import numpy as np
import jax
import jax.numpy as jnp
from jax import lax
from jax.experimental import pallas as pl
from jax.experimental.pallas import tpu as pltpu
from jax.experimental.pallas import tpu_sc as plsc

N = 1000
P = 1024
NV = P // 16
TOPP = 64
TOPN = 50
MIN32 = np.int32(-(1 << 31))

mesh = plsc.VectorSubcoreMesh(core_axis_name="c", subcore_axis_name="s")
NW = 32


def _make_sc_sort(B, RI=4, unroll=2):
    rows_per_w = B // NW
    groups = rows_per_w // RI

    @functools.partial(
        pl.kernel,
        mesh=mesh,
        out_type=[
            jax.ShapeDtypeStruct((B, TOPP), jnp.int32),
            jax.ShapeDtypeStruct((B, N), jnp.float32),
        ],
        scratch_types=[
            pltpu.VMEM((RI * P,), jnp.float32),
            pltpu.VMEM((RI * P,), jnp.int32),
            pltpu.VMEM((RI * P,), jnp.int32),
            pltpu.VMEM((RI * P,), jnp.int32),
            pltpu.VMEM((RI * P,), jnp.int32),
            pltpu.VMEM((RI * 256,), jnp.int32),
            pltpu.VMEM((RI * 16,), jnp.int32),
        ],
        compiler_params=pltpu.CompilerParams(
            needs_layout_passes=False, use_tc_tiling_on_sc=False),
    )
    def sc_sort(x_hbm, oidx_hbm, oconf_hbm, vbuf, ka, ia, kb, ib, bins, tots):
        wid = lax.axis_index("s") * 2 + lax.axis_index("c")
        row0 = wid * rows_per_w

        lanes = lax.iota(jnp.int32, 16)
        zero = lanes * 0
        one = zero + 1
        ninf = zero.astype(jnp.float32) + jnp.float32(-jnp.inf)
        lane_is_last = lanes == 15

        def do_group(g, _):
            grow = row0 + g * RI
            for rr in range(RI):
                vbuf[pl.ds(rr * P + P - 32, 16)] = ninf
                vbuf[pl.ds(rr * P + P - 16, 16)] = ninf
            for rr in range(RI):
                pltpu.sync_copy(x_hbm.at[grow + rr],
                                vbuf.at[pl.ds(rr * P, N)])

            def init_j(j, _):
                for rr in range(RI):
                    o = rr * P + j * 16
                    f = vbuf[pl.ds(o, 16)]
                    u = plsc.bitcast(f, jnp.int32)
                    m = lax.shift_right_arithmetic(u, 31) | MIN32
                    ka[pl.ds(o, 16)] = ~(u ^ m)
                    ia[pl.ds(o, 16)] = lanes + j * 16
                return 0
            lax.fori_loop(0, NV, init_j, 0, unroll=unroll)

            def radix_pass(shift, src_k, src_i, dst_k, dst_i):
                def clr(t, _):
                    for rr in range(RI):
                        bins[pl.ds(rr * 256 + t * 16, 16)] = zero
                    return 0
                lax.fori_loop(0, 16, clr, 0, unroll=unroll)

                def hist(j, _):
                    for rr in range(RI):
                        k = src_k[pl.ds(rr * P + j * 16, 16)]
                        d = lax.shift_right_logical(k, shift) & 255
                        plsc.addupdate_scatter(bins, [d + rr * 256], one)
                    return 0
                lax.fori_loop(0, NV, hist, 0, unroll=unroll)

                def scang(t, _):
                    for rr in range(RI):
                        v = bins[pl.ds(rr * 256 + t * 16, 16)]
                        c = plsc.cumsum(v)
                        bins[pl.ds(rr * 256 + t * 16, 16)] = c - v
                        plsc.store_scatter(tots, [zero + (rr * 16 + t)], c,
                                           mask=lane_is_last)
                    return 0
                lax.fori_loop(0, 16, scang, 0)
                for rr in range(RI):
                    tv = tots[pl.ds(rr * 16, 16)]
                    tots[pl.ds(rr * 16, 16)] = plsc.cumsum(tv) - tv

                def addb(t, _):
                    for rr in range(RI):
                        te = plsc.load_gather(tots, [zero + (rr * 16 + t)])
                        o = rr * 256 + t * 16
                        bins[pl.ds(o, 16)] = bins[pl.ds(o, 16)] + te
                    return 0
                lax.fori_loop(0, 16, addb, 0)

                def perm(j, _):
                    for rr in range(RI):
                        o = rr * P + j * 16
                        k = src_k[pl.ds(o, 16)]
                        d = (lax.shift_right_logical(k, shift) & 255) + rr * 256
                        base = plsc.load_gather(bins, [d])
                        sc, _unused = plsc.scan_count(d)
                        ofs = base + sc - 1 + rr * P
                        plsc.store_scatter(dst_k, [ofs], k)
                        plsc.store_scatter(dst_i, [ofs], src_i[pl.ds(o, 16)])
                        plsc.addupdate_scatter(bins, [d], one)
                    return 0
                lax.fori_loop(0, NV, perm, 0, unroll=unroll)

            radix_pass(0, ka, ia, kb, ib)
            radix_pass(8, kb, ib, ka, ia)
            radix_pass(16, ka, ia, kb, ib)
            radix_pass(24, kb, ib, ka, ia)

            def fin_j(j, _):
                for rr in range(RI):
                    o = rr * P + j * 16
                    kd = ka[pl.ds(o, 16)]
                    k0 = ~kd
                    m = (~lax.shift_right_arithmetic(k0, 31)) | MIN32
                    vbuf[pl.ds(o, 16)] = plsc.bitcast(k0 ^ m, jnp.float32)
                return 0
            lax.fori_loop(0, NV, fin_j, 0, unroll=unroll)

            for rr in range(RI):
                pltpu.sync_copy(vbuf.at[pl.ds(rr * P, N)],
                                oconf_hbm.at[grow + rr])
                pltpu.sync_copy(ia.at[pl.ds(rr * P, TOPP)],
                                oidx_hbm.at[grow + rr])
            return 0

        lax.fori_loop(0, groups, do_group, 0)

    return sc_sort


_sc_sort = _make_sc_sort(16384, RI=4, unroll=2)


@jax.jit
def kernel(x, label_ids):
    top_idx, conf = _sc_sort(x)
    top_label = jnp.take(label_ids, top_idx[:, :TOPN])
    return (top_label, conf)


# speedup vs baseline: 2.4997x; 2.4997x over previous
import functools
import numpy as np
import jax
import jax.numpy as jnp
from jax import lax
from jax.experimental import pallas as pl
from jax.experimental.pallas import tpu as pltpu
from jax.experimental.pallas import tpu_sc as plsc

N = 1000
P = 1024
NV = P // 16
TOPP = 64
TOPN = 50
MIN32 = np.int32(-(1 << 31))

mesh = plsc.VectorSubcoreMesh(core_axis_name="c", subcore_axis_name="s")
NW = 32


def _make_sc_sort(B, RI=4, unroll=2):
    rows_per_w = B // NW
    groups = rows_per_w // RI

    @functools.partial(
        pl.kernel,
        mesh=mesh,
        out_type=[
            jax.ShapeDtypeStruct((B, TOPP), jnp.int32),
            jax.ShapeDtypeStruct((B, N), jnp.float32),
        ],
        scratch_types=[
            pltpu.VMEM((RI * P,), jnp.float32),
            pltpu.VMEM((RI * P,), jnp.int32),
            pltpu.VMEM((RI * P,), jnp.int32),
            pltpu.VMEM((RI * P,), jnp.int32),
            pltpu.VMEM((RI * P,), jnp.int32),
            pltpu.VMEM((RI * 256,), jnp.int32),
            pltpu.VMEM((RI * 16,), jnp.int32),
            pltpu.VMEM((P,), jnp.int32),
            pltpu.VMEM((RI * TOPP,), jnp.int32),
        ],
        compiler_params=pltpu.CompilerParams(
            needs_layout_passes=False, use_tc_tiling_on_sc=False),
    )
    def sc_sort(x_hbm, lab_hbm, oidx_hbm, oconf_hbm, vbuf, ka, ia, kb, ib,
                bins, tots, labv, olab):
        wid = lax.axis_index("s") * 2 + lax.axis_index("c")
        row0 = wid * rows_per_w

        lanes = lax.iota(jnp.int32, 16)
        zero = lanes * 0
        one = zero + 1
        ninf = zero.astype(jnp.float32) + jnp.float32(-jnp.inf)
        lane_is_last = lanes == 15

        pltpu.sync_copy(lab_hbm, labv.at[pl.ds(0, N)])

        def do_group(g, _):
            grow = row0 + g * RI
            for rr in range(RI):
                vbuf[pl.ds(rr * P + P - 32, 16)] = ninf
                vbuf[pl.ds(rr * P + P - 16, 16)] = ninf
            for rr in range(RI):
                pltpu.sync_copy(x_hbm.at[grow + rr],
                                vbuf.at[pl.ds(rr * P, N)])

            def init_j(j, _):
                for rr in range(RI):
                    o = rr * P + j * 16
                    f = vbuf[pl.ds(o, 16)]
                    u = plsc.bitcast(f, jnp.int32)
                    m = lax.shift_right_arithmetic(u, 31) | MIN32
                    ka[pl.ds(o, 16)] = ~(u ^ m)
                    ia[pl.ds(o, 16)] = lanes + j * 16
                return 0
            lax.fori_loop(0, NV, init_j, 0, unroll=unroll)

            def radix_pass(shift, src_k, src_i, dst_k, dst_i):
                def clr(t, _):
                    for rr in range(RI):
                        bins[pl.ds(rr * 256 + t * 16, 16)] = zero
                    return 0
                lax.fori_loop(0, 16, clr, 0, unroll=unroll)

                def hist(j, _):
                    for rr in range(RI):
                        k = src_k[pl.ds(rr * P + j * 16, 16)]
                        d = lax.shift_right_logical(k, shift) & 255
                        plsc.addupdate_scatter(bins, [d + rr * 256], one)
                    return 0
                lax.fori_loop(0, NV, hist, 0, unroll=unroll)

                def scang(t, _):
                    for rr in range(RI):
                        v = bins[pl.ds(rr * 256 + t * 16, 16)]
                        c = plsc.cumsum(v)
                        bins[pl.ds(rr * 256 + t * 16, 16)] = c - v
                        plsc.store_scatter(tots, [zero + (rr * 16 + t)], c,
                                           mask=lane_is_last)
                    return 0
                lax.fori_loop(0, 16, scang, 0)
                for rr in range(RI):
                    tv = tots[pl.ds(rr * 16, 16)]
                    tots[pl.ds(rr * 16, 16)] = plsc.cumsum(tv) - tv

                def addb(t, _):
                    for rr in range(RI):
                        te = plsc.load_gather(tots, [zero + (rr * 16 + t)])
                        o = rr * 256 + t * 16
                        bins[pl.ds(o, 16)] = bins[pl.ds(o, 16)] + te
                    return 0
                lax.fori_loop(0, 16, addb, 0)

                def perm(j, _):
                    for rr in range(RI):
                        o = rr * P + j * 16
                        k = src_k[pl.ds(o, 16)]
                        d = (lax.shift_right_logical(k, shift) & 255) + rr * 256
                        base = plsc.load_gather(bins, [d])
                        sc, _unused = plsc.scan_count(d)
                        ofs = base + sc - 1 + rr * P
                        plsc.store_scatter(dst_k, [ofs], k)
                        plsc.store_scatter(dst_i, [ofs], src_i[pl.ds(o, 16)])
                        plsc.addupdate_scatter(bins, [d], one)
                    return 0
                lax.fori_loop(0, NV, perm, 0, unroll=unroll)

            radix_pass(0, ka, ia, kb, ib)
            radix_pass(8, kb, ib, ka, ia)
            radix_pass(16, ka, ia, kb, ib)
            radix_pass(24, kb, ib, ka, ia)

            def fin_j(j, _):
                for rr in range(RI):
                    o = rr * P + j * 16
                    kd = ka[pl.ds(o, 16)]
                    k0 = ~kd
                    m = (~lax.shift_right_arithmetic(k0, 31)) | MIN32
                    vbuf[pl.ds(o, 16)] = plsc.bitcast(k0 ^ m, jnp.float32)
                return 0
            lax.fori_loop(0, NV, fin_j, 0, unroll=unroll)

            for rr in range(RI):
                for q in range(TOPP // 16):
                    iv = ia[pl.ds(rr * P + q * 16, 16)]
                    olab[pl.ds(rr * TOPP + q * 16, 16)] = (
                        plsc.load_gather(labv, [iv]))

            for rr in range(RI):
                pltpu.sync_copy(vbuf.at[pl.ds(rr * P, N)],
                                oconf_hbm.at[grow + rr])
                pltpu.sync_copy(olab.at[pl.ds(rr * TOPP, TOPP)],
                                oidx_hbm.at[grow + rr])
            return 0

        lax.fori_loop(0, groups, do_group, 0)

    return sc_sort


_sc_sort = _make_sc_sort(16384, RI=4, unroll=2)


@jax.jit
def kernel(x, label_ids):
    top_lab, conf = _sc_sort(x, label_ids)
    return (top_lab[:, :TOPN], conf)


# speedup vs baseline: 2.6664x; 1.0667x over previous
import functools
import numpy as np
import jax
import jax.numpy as jnp
from jax import lax
from jax.experimental import pallas as pl
from jax.experimental.pallas import tpu as pltpu
from jax.experimental.pallas import tpu_sc as plsc

N = 1000
P = 1024
NV = P // 16
TOPP = 64
TOPN = 50
MIN32 = np.int32(-(1 << 31))

mesh = plsc.VectorSubcoreMesh(core_axis_name="c", subcore_axis_name="s")
NW = 32


def _make_sc_sort(B, RI=4, unroll=2):
    rows_per_w = B // NW
    groups = rows_per_w // RI
    T = groups // 2

    @functools.partial(
        pl.kernel,
        mesh=mesh,
        out_type=[
            jax.ShapeDtypeStruct((B * TOPP,), jnp.int32),
            jax.ShapeDtypeStruct((B * P,), jnp.float32),
        ],
        scratch_types=[
            pltpu.VMEM((RI * P,), jnp.float32),
            pltpu.VMEM((RI * P,), jnp.float32),
            pltpu.VMEM((RI * P,), jnp.float32),
            pltpu.VMEM((RI * P,), jnp.float32),
            pltpu.VMEM((RI * TOPP,), jnp.int32),
            pltpu.VMEM((RI * TOPP,), jnp.int32),
            pltpu.VMEM((RI * P,), jnp.int32),
            pltpu.VMEM((RI * P,), jnp.int32),
            pltpu.VMEM((RI * P,), jnp.int32),
            pltpu.VMEM((RI * P,), jnp.int32),
            pltpu.VMEM((RI * 256,), jnp.int32),
            pltpu.VMEM((RI * 16,), jnp.int32),
            pltpu.VMEM((P,), jnp.int32),
            pltpu.SemaphoreType.DMA,
            pltpu.SemaphoreType.DMA,
            pltpu.SemaphoreType.DMA,
        ],
        compiler_params=pltpu.CompilerParams(
            needs_layout_passes=False, use_tc_tiling_on_sc=False),
    )
    def sc_sort(x_hbm, lab_hbm, oidx_hbm, oconf_hbm,
                vin0, vin1, vout0, vout1, olab0, olab1,
                ka, ia, kb, ib, bins, tots, labv, sin0, sin1, sout):
        wid = lax.axis_index("s") * 2 + lax.axis_index("c")
        row0 = wid * rows_per_w

        lanes = lax.iota(jnp.int32, 16)
        zero = lanes * 0
        one = zero + 1
        ninf = zero.astype(jnp.float32) + jnp.float32(-jnp.inf)
        lane_is_last = lanes == 15

        pltpu.sync_copy(lab_hbm, labv.at[pl.ds(0, N)])
        for vin in (vin0, vin1):
            for rr in range(RI):
                vin[pl.ds(rr * P + P - 32, 16)] = ninf
                vin[pl.ds(rr * P + P - 16, 16)] = ninf

        def fire_in(vin, sem, grow):
            for rr in range(RI):
                pltpu.make_async_copy(
                    x_hbm.at[grow + rr], vin.at[pl.ds(rr * P, N)], sem
                ).start()

        def drain_in(vin, sem, grow):
            for rr in range(RI):
                pltpu.make_async_copy(
                    x_hbm.at[grow + rr], vin.at[pl.ds(rr * P, N)], sem
                ).wait()

        def fire_out(vout, olab, g):
            pltpu.make_async_copy(
                vout, oconf_hbm.at[pl.ds((row0 + g * RI) * P, RI * P)], sout
            ).start()
            pltpu.make_async_copy(
                olab, oidx_hbm.at[pl.ds((row0 + g * RI) * TOPP, RI * TOPP)],
                sout,
            ).start()

        def drain_out(vout, olab, g):
            pltpu.make_async_copy(
                vout, oconf_hbm.at[pl.ds((row0 + g * RI) * P, RI * P)], sout
            ).wait()
            pltpu.make_async_copy(
                olab, oidx_hbm.at[pl.ds((row0 + g * RI) * TOPP, RI * TOPP)],
                sout,
            ).wait()

        def compute(vin, vout, olab):
            def init_j(j, _):
                for rr in range(RI):
                    o = rr * P + j * 16
                    f = vin[pl.ds(o, 16)]
                    u = plsc.bitcast(f, jnp.int32)
                    m = lax.shift_right_arithmetic(u, 31) | MIN32
                    ka[pl.ds(o, 16)] = ~(u ^ m)
                    ia[pl.ds(o, 16)] = lanes + j * 16
                return 0
            lax.fori_loop(0, NV, init_j, 0, unroll=unroll)

            def radix_pass(shift, src_k, src_i, dst_k, dst_i):
                def clr(t, _):
                    for rr in range(RI):
                        bins[pl.ds(rr * 256 + t * 16, 16)] = zero
                    return 0
                lax.fori_loop(0, 16, clr, 0, unroll=unroll)

                def hist(j, _):
                    for rr in range(RI):
                        k = src_k[pl.ds(rr * P + j * 16, 16)]
                        d = lax.shift_right_logical(k, shift) & 255
                        plsc.addupdate_scatter(bins, [d + rr * 256], one)
                    return 0
                lax.fori_loop(0, NV, hist, 0, unroll=unroll)

                def scang(t, _):
                    for rr in range(RI):
                        v = bins[pl.ds(rr * 256 + t * 16, 16)]
                        c = plsc.cumsum(v)
                        bins[pl.ds(rr * 256 + t * 16, 16)] = c - v
                        plsc.store_scatter(tots, [zero + (rr * 16 + t)], c,
                                           mask=lane_is_last)
                    return 0
                lax.fori_loop(0, 16, scang, 0)
                for rr in range(RI):
                    tv = tots[pl.ds(rr * 16, 16)]
                    tots[pl.ds(rr * 16, 16)] = plsc.cumsum(tv) - tv

                def addb(t, _):
                    for rr in range(RI):
                        te = plsc.load_gather(tots, [zero + (rr * 16 + t)])
                        o = rr * 256 + t * 16
                        bins[pl.ds(o, 16)] = bins[pl.ds(o, 16)] + te
                    return 0
                lax.fori_loop(0, 16, addb, 0)

                def perm(j, _):
                    for rr in range(RI):
                        o = rr * P + j * 16
                        k = src_k[pl.ds(o, 16)]
                        d = (lax.shift_right_logical(k, shift) & 255) + rr * 256
                        base = plsc.load_gather(bins, [d])
                        sc, _unused = plsc.scan_count(d)
                        ofs = base + sc - 1 + rr * P
                        plsc.store_scatter(dst_k, [ofs], k)
                        plsc.store_scatter(dst_i, [ofs], src_i[pl.ds(o, 16)])
                        plsc.addupdate_scatter(bins, [d], one)
                    return 0
                lax.fori_loop(0, NV, perm, 0, unroll=unroll)

            radix_pass(0, ka, ia, kb, ib)
            radix_pass(8, kb, ib, ka, ia)
            radix_pass(16, ka, ia, kb, ib)
            radix_pass(24, kb, ib, ka, ia)

            def fin_j(j, _):
                for rr in range(RI):
                    o = rr * P + j * 16
                    kd = ka[pl.ds(o, 16)]
                    k0 = ~kd
                    m = (~lax.shift_right_arithmetic(k0, 31)) | MIN32
                    vout[pl.ds(o, 16)] = plsc.bitcast(k0 ^ m, jnp.float32)
                return 0
            lax.fori_loop(0, NV, fin_j, 0, unroll=unroll)

            for rr in range(RI):
                for q in range(TOPP // 16):
                    iv = ia[pl.ds(rr * P + q * 16, 16)]
                    olab[pl.ds(rr * TOPP + q * 16, 16)] = (
                        plsc.load_gather(labv, [iv]))

        fire_in(vin0, sin0, row0)

        def body(t, _):
            ga = 2 * t
            gb = ga + 1
            rowa = row0 + ga * RI
            rowb = rowa + RI

            drain_in(vin0, sin0, rowa)
            fire_in(vin1, sin1, rowb)
            compute(vin0, vout0, olab0)
            fire_out(vout0, olab0, ga)

            drain_in(vin1, sin1, rowb)

            @pl.when(t < T - 1)
            def _():
                fire_in(vin0, sin0, rowb + RI)

            compute(vin1, vout1, olab1)
            drain_out(vout0, olab0, ga)
            fire_out(vout1, olab1, gb)
            drain_out(vout1, olab1, gb)
            return 0

        lax.fori_loop(0, T, body, 0)

    return sc_sort


_B = 16384
_sc_sort = _make_sc_sort(_B, RI=4, unroll=2)


@jax.jit
def kernel(x, label_ids):
    top_flat, conf_flat = _sc_sort(x, label_ids)
    top_label = top_flat.reshape(_B, TOPP)[:, :TOPN]
    conf = conf_flat.reshape(_B, P)[:, :N]
    return (top_label, conf)


# speedup vs baseline: 2.9068x; 1.0902x over previous
import functools
import numpy as np
import jax
import jax.numpy as jnp
from jax import lax
from jax.experimental import pallas as pl
from jax.experimental.pallas import tpu as pltpu
from jax.experimental.pallas import tpu_sc as plsc

N = 1000
P = 1024
NV = P // 16
TOPP = 64
TOPN = 50
MIN32 = np.int32(-(1 << 31))

mesh = plsc.VectorSubcoreMesh(core_axis_name="c", subcore_axis_name="s")
NW = 32


def _make_sc_sort(B, RI=4, unroll=2):
    rows_per_w = B // NW
    groups = rows_per_w // RI
    T = groups // 2

    @functools.partial(
        pl.kernel,
        mesh=mesh,
        out_type=[
            jax.ShapeDtypeStruct((B * TOPP,), jnp.int32),
            jax.ShapeDtypeStruct((B * P,), jnp.float32),
        ],
        scratch_types=[
            pltpu.VMEM((RI * P,), jnp.float32),
            pltpu.VMEM((RI * P,), jnp.float32),
            pltpu.VMEM((RI * P,), jnp.float32),
            pltpu.VMEM((RI * P,), jnp.float32),
            pltpu.VMEM((RI * TOPP,), jnp.int32),
            pltpu.VMEM((RI * TOPP,), jnp.int32),
            pltpu.VMEM((RI * P,), jnp.int32),
            pltpu.VMEM((RI * P,), jnp.int32),
            pltpu.VMEM((RI * P,), jnp.int32),
            pltpu.VMEM((RI * P,), jnp.int32),
            pltpu.VMEM((RI * 256,), jnp.int32),
            pltpu.VMEM((RI * 16,), jnp.int32),
            pltpu.VMEM((P,), jnp.int32),
            pltpu.SemaphoreType.DMA,
            pltpu.SemaphoreType.DMA,
            pltpu.SemaphoreType.DMA,
        ],
        compiler_params=pltpu.CompilerParams(
            needs_layout_passes=False, use_tc_tiling_on_sc=False),
    )
    def sc_sort(x_hbm, lab_hbm, oidx_hbm, oconf_hbm,
                vin0, vin1, vout0, vout1, olab0, olab1,
                ka, ia, kb, ib, bins, tots, labv, sin0, sin1, sout):
        wid = lax.axis_index("s") * 2 + lax.axis_index("c")
        row0 = wid * rows_per_w

        lanes = lax.iota(jnp.int32, 16)
        zero = lanes * 0
        one = zero + 1
        ninf = zero.astype(jnp.float32) + jnp.float32(-jnp.inf)
        lane_is_last = lanes == 15

        pltpu.sync_copy(lab_hbm, labv.at[pl.ds(0, N)])
        for vin in (vin0, vin1):
            for rr in range(RI):
                vin[pl.ds(rr * P + P - 32, 16)] = ninf
                vin[pl.ds(rr * P + P - 16, 16)] = ninf

        def fire_in(vin, sem, grow):
            for rr in range(RI):
                pltpu.make_async_copy(
                    x_hbm.at[grow + rr], vin.at[pl.ds(rr * P, N)], sem
                ).start()

        def drain_in(vin, sem, grow):
            for rr in range(RI):
                pltpu.make_async_copy(
                    x_hbm.at[grow + rr], vin.at[pl.ds(rr * P, N)], sem
                ).wait()

        def fire_out(vout, olab, g):
            pltpu.make_async_copy(
                vout, oconf_hbm.at[pl.ds((row0 + g * RI) * P, RI * P)], sout
            ).start()
            pltpu.make_async_copy(
                olab, oidx_hbm.at[pl.ds((row0 + g * RI) * TOPP, RI * TOPP)],
                sout,
            ).start()

        def drain_out(vout, olab, g):
            pltpu.make_async_copy(
                vout, oconf_hbm.at[pl.ds((row0 + g * RI) * P, RI * P)], sout
            ).wait()
            pltpu.make_async_copy(
                olab, oidx_hbm.at[pl.ds((row0 + g * RI) * TOPP, RI * TOPP)],
                sout,
            ).wait()

        def key0(f):
            u = plsc.bitcast(f, jnp.int32)
            m = lax.shift_right_arithmetic(u, 31) | MIN32
            return ~(u ^ m)

        def unkey(k):
            k0 = ~k
            m = (~lax.shift_right_arithmetic(k0, 31)) | MIN32
            return plsc.bitcast(k0 ^ m, jnp.float32)

        def compute(vin, vout, olab):
            def scan_bins():
                def scang(t, _):
                    for rr in range(RI):
                        v = bins[pl.ds(rr * 256 + t * 16, 16)]
                        c = plsc.cumsum(v)
                        bins[pl.ds(rr * 256 + t * 16, 16)] = c - v
                        plsc.store_scatter(tots, [zero + (rr * 16 + t)], c,
                                           mask=lane_is_last)
                    return 0
                lax.fori_loop(0, 16, scang, 0)
                for rr in range(RI):
                    tv = tots[pl.ds(rr * 16, 16)]
                    tots[pl.ds(rr * 16, 16)] = plsc.cumsum(tv) - tv

                def addb(t, _):
                    for rr in range(RI):
                        te = plsc.load_gather(tots, [zero + (rr * 16 + t)])
                        o = rr * 256 + t * 16
                        bins[pl.ds(o, 16)] = bins[pl.ds(o, 16)] + te
                    return 0
                lax.fori_loop(0, 16, addb, 0)

            def clr_bins():
                def clr(t, _):
                    for rr in range(RI):
                        bins[pl.ds(rr * 256 + t * 16, 16)] = zero
                    return 0
                lax.fori_loop(0, 16, clr, 0, unroll=unroll)

            clr_bins()

            def hist0(j, _):
                for rr in range(RI):
                    k = key0(vin[pl.ds(rr * P + j * 16, 16)])
                    plsc.addupdate_scatter(bins, [(k & 255) + rr * 256], one)
                return 0
            lax.fori_loop(0, NV, hist0, 0, unroll=unroll)
            scan_bins()

            def perm0(j, _):
                iv = lanes + j * 16
                for rr in range(RI):
                    k = key0(vin[pl.ds(rr * P + j * 16, 16)])
                    d = (k & 255) + rr * 256
                    base = plsc.load_gather(bins, [d])
                    sc, _unused = plsc.scan_count(d)
                    ofs = base + sc + (rr * P - 1)
                    plsc.store_scatter(kb, [ofs], k)
                    plsc.store_scatter(ib, [ofs], iv)
                    plsc.addupdate_scatter(bins, [d], one)
                return 0
            lax.fori_loop(0, NV, perm0, 0, unroll=unroll)

            def radix_pass(shift, src_k, src_i, dst_k, dst_i):
                clr_bins()

                def hist(j, _):
                    for rr in range(RI):
                        k = src_k[pl.ds(rr * P + j * 16, 16)]
                        d = lax.shift_right_logical(k, shift) & 255
                        plsc.addupdate_scatter(bins, [d + rr * 256], one)
                    return 0
                lax.fori_loop(0, NV, hist, 0, unroll=unroll)
                scan_bins()

                def perm(j, _):
                    for rr in range(RI):
                        o = rr * P + j * 16
                        k = src_k[pl.ds(o, 16)]
                        d = (lax.shift_right_logical(k, shift) & 255) + rr * 256
                        base = plsc.load_gather(bins, [d])
                        sc, _unused = plsc.scan_count(d)
                        ofs = base + sc + (rr * P - 1)
                        plsc.store_scatter(dst_k, [ofs], k)
                        plsc.store_scatter(dst_i, [ofs], src_i[pl.ds(o, 16)])
                        plsc.addupdate_scatter(bins, [d], one)
                    return 0
                lax.fori_loop(0, NV, perm, 0, unroll=unroll)

            radix_pass(8, kb, ib, ka, ia)
            radix_pass(16, ka, ia, kb, ib)

            clr_bins()

            def hist3(j, _):
                for rr in range(RI):
                    k = kb[pl.ds(rr * P + j * 16, 16)]
                    d = lax.shift_right_logical(k, 24)
                    plsc.addupdate_scatter(bins, [d + rr * 256], one)
                return 0
            lax.fori_loop(0, NV, hist3, 0, unroll=unroll)
            scan_bins()

            def perm3(j, _):
                for rr in range(RI):
                    o = rr * P + j * 16
                    k = kb[pl.ds(o, 16)]
                    d = lax.shift_right_logical(k, 24) + rr * 256
                    base = plsc.load_gather(bins, [d])
                    sc, _unused = plsc.scan_count(d)
                    ofs = base + sc + (rr * P - 1)
                    plsc.store_scatter(vout, [ofs], unkey(k))
                    plsc.store_scatter(ia, [ofs], ib[pl.ds(o, 16)])
                    plsc.addupdate_scatter(bins, [d], one)
                return 0
            lax.fori_loop(0, NV, perm3, 0, unroll=unroll)

            for rr in range(RI):
                for q in range(TOPP // 16):
                    iv = ia[pl.ds(rr * P + q * 16, 16)]
                    olab[pl.ds(rr * TOPP + q * 16, 16)] = (
                        plsc.load_gather(labv, [iv]))

        fire_in(vin0, sin0, row0)

        def body(t, _):
            ga = 2 * t
            gb = ga + 1
            rowa = row0 + ga * RI
            rowb = rowa + RI

            drain_in(vin0, sin0, rowa)
            fire_in(vin1, sin1, rowb)
            compute(vin0, vout0, olab0)
            fire_out(vout0, olab0, ga)

            drain_in(vin1, sin1, rowb)

            @pl.when(t < T - 1)
            def _():
                fire_in(vin0, sin0, rowb + RI)

            compute(vin1, vout1, olab1)
            drain_out(vout0, olab0, ga)
            fire_out(vout1, olab1, gb)
            drain_out(vout1, olab1, gb)
            return 0

        lax.fori_loop(0, T, body, 0)

    return sc_sort


_B = 16384
_sc_sort = _make_sc_sort(_B, RI=4, unroll=2)


@jax.jit
def kernel(x, label_ids):
    top_flat, conf_flat = _sc_sort(x, label_ids)
    top_label = top_flat.reshape(_B, TOPP)[:, :TOPN]
    conf = conf_flat.reshape(_B, P)[:, :N]
    return (top_label, conf)
